# baseline (speedup 1.0000x reference)
import functools

import jax
import jax.numpy as jnp
from jax import lax
from jax.experimental import pallas as pl
from jax.experimental.pallas import tpu as pltpu
from jax.experimental.pallas import tpu_sc as plsc

N_NODES = 10000
DEG = 16
D_FEAT = 128
N_RADIAL = 8
TP_CH = 32
SH_DIM = 8
MSG_DIM = D_FEAT + TP_CH * SH_DIM
E_TOTAL = N_NODES * DEG

B_NODES = 200
B_EDGES = B_NODES * DEG
N_BLOCKS = N_NODES // B_NODES

_SQ3 = 3.0 ** 0.5
_SQ15 = 15.0 ** 0.5
_SQ5H = (5.0 ** 0.5) / 2.0
_SQ15H = _SQ15 / 2.0
_SCALE = 0.25


def _tc_body(v_ref, r_ref, feats_ref, w2_ref, t_ref, wr_ref, br_ref, g_ref,
             b_ref, out_ref):
    feats = feats_ref[...]
    p2 = jnp.dot(feats, w2_ref[...], preferred_element_type=jnp.float32)
    p2e = jnp.reshape(jnp.broadcast_to(p2[:, None, :], (B_NODES, DEG, 2 * D_FEAT)),
                      (B_EDGES, 2 * D_FEAT))
    fe = jnp.reshape(jnp.broadcast_to(feats[:, None, :], (B_NODES, DEG, D_FEAT)),
                     (B_EDGES, D_FEAT))
    v = v_ref[...]
    x = v[:, 0:1]
    y = v[:, 1:2]
    z = v[:, 2:3]
    inv = 1.0 / (jnp.sqrt(x * x + y * y + z * z) + 1e-9)
    x = x * inv
    y = y * inv
    z = z * inv
    sh8 = jnp.concatenate([
        _SQ3 * x, _SQ3 * y, _SQ3 * z,
        _SQ15 * x * y, _SQ15 * y * z, _SQ5H * (3.0 * z * z - 1.0),
        _SQ15 * x * z, _SQ15H * (x * x - y * y),
    ], axis=1)
    shm = jnp.dot(sh8, t_ref[...], preferred_element_type=jnp.float32)
    radial = jnp.dot(r_ref[...], wr_ref[...],
                     preferred_element_type=jnp.float32) + br_ref[...]
    mu = jnp.mean(radial, axis=1, keepdims=True)
    d = radial - mu
    var = jnp.mean(d * d, axis=1, keepdims=True)
    radial = (d * lax.rsqrt(var + 1e-6) * g_ref[...] + b_ref[...]) * _SCALE
    out_ref[...] = jnp.concatenate(
        [fe * radial[:, :D_FEAT], p2e * shm * radial[:, D_FEAT:]], axis=1)


def _messages_tc(vecs2, rad2, node_feats, w2, t, wr, br2, g2, b2):
    return pl.pallas_call(
        _tc_body,
        grid=(N_BLOCKS,),
        in_specs=[
            pl.BlockSpec((B_EDGES, 3), lambda i: (i, 0)),
            pl.BlockSpec((B_EDGES, N_RADIAL), lambda i: (i, 0)),
            pl.BlockSpec((B_NODES, D_FEAT), lambda i: (i, 0)),
            pl.BlockSpec((D_FEAT, 2 * D_FEAT), lambda i: (0, 0)),
            pl.BlockSpec((SH_DIM, 2 * D_FEAT), lambda i: (0, 0)),
            pl.BlockSpec((N_RADIAL, MSG_DIM), lambda i: (0, 0)),
            pl.BlockSpec((1, MSG_DIM), lambda i: (0, 0)),
            pl.BlockSpec((1, MSG_DIM), lambda i: (0, 0)),
            pl.BlockSpec((1, MSG_DIM), lambda i: (0, 0)),
        ],
        out_specs=pl.BlockSpec((B_EDGES, MSG_DIM), lambda i: (i, 0)),
        out_shape=jax.ShapeDtypeStruct((E_TOTAL, MSG_DIM), jnp.float32),
    )(vecs2, rad2, node_feats, w2, t, wr, br2, g2, b2)


ACC_ROWS = 160
W_BIG = 160
W_SMALL = 152
N_BIG = 34
CAP = 512
IDS_BLOCK = 640
EBUF = CAP + IDS_BLOCK + 32
N_SCAN = E_TOTAL // IDS_BLOCK
GCHUNK = 32


@functools.cache
def _make_scatter_sc():
    mesh = plsc.VectorSubcoreMesh(core_axis_name="c", subcore_axis_name="s",
                                  num_cores=2, num_subcores=16)
    return pl.kernel(
        _sc_body,
        out_type=jax.ShapeDtypeStruct((N_NODES, MSG_DIM), jnp.float32),
        mesh=mesh,
        compiler_params=pltpu.CompilerParams(needs_layout_passes=False),
        scratch_types=[
            pltpu.VMEM((ACC_ROWS, MSG_DIM), jnp.float32),
            pltpu.VMEM((GCHUNK, MSG_DIM), jnp.float32),
            pltpu.VMEM((EBUF,), jnp.int32),
            pltpu.VMEM((EBUF,), jnp.int32),
            pltpu.VMEM((IDS_BLOCK,), jnp.int32),
        ],
    )


def _sc_body(msgs_hbm, recv_hbm, out_hbm, acc, mbuf, ebuf, rbuf, idbuf):
    cid = lax.axis_index("c")
    sid = lax.axis_index("s")
    t = sid * 2 + cid

    iota16 = lax.iota(jnp.int32, 16)
    zero16f = jnp.zeros((16,), jnp.float32)
    zero16i = jnp.zeros((16,), jnp.int32)
    colv = [cg * 16 + iota16 for cg in range(MSG_DIM // 16)]

    def drain(n):
        nch = (n + (GCHUNK - 1)) // GCHUNK

        def dbody(dk, carry):
            pltpu.sync_copy(msgs_hbm.at[ebuf.at[pl.ds(dk * GCHUNK, GCHUNK)]],
                            mbuf)
            nrows = jnp.minimum(GCHUNK, n - dk * GCHUNK)

            def rbody(r, c2):
                fr = jnp.full((16,), dk * GCHUNK + r, jnp.int32)
                rd = jnp.max(plsc.load_gather(rbuf, [fr]))
                for cg in range(MSG_DIM // 16):
                    vals = mbuf[r, pl.ds(cg * 16, 16)]
                    plsc.addupdate(acc.at[rd, pl.ds(cg * 16, 16)], vals)
                return c2

            lax.fori_loop(0, nrows, rbody, 0)
            return carry

        lax.fori_loop(0, nch, dbody, 0)

    def pbody(p, pcarry):
        w = t + 32 * p
        w0 = jnp.where(w < N_BIG, w * W_BIG,
                       N_BIG * W_BIG + (w - N_BIG) * W_SMALL).astype(jnp.int32)
        w0 = pl.multiple_of(w0, 8)
        wlim = jnp.where(w < N_BIG, W_BIG, W_SMALL).astype(jnp.int32)

        def zbody(r, zc):
            for cg in range(MSG_DIM // 16):
                acc[r, pl.ds(cg * 16, 16)] = zero16f
            return zc

        lax.fori_loop(0, ACC_ROWS, zbody, 0)

        def ezbody(k, zc):
            plsc.store_scatter(ebuf, [k * 16 + iota16], zero16i)
            return zc

        lax.fori_loop(0, EBUF // 16, ezbody, 0)

        def sbody(o, cnt):
            pltpu.sync_copy(recv_hbm.at[pl.ds(o * IDS_BLOCK, IDS_BLOCK)],
                            idbuf)

            def cbody(ich, cnt2):
                rv = plsc.load_gather(idbuf, [ich * 16 + iota16])
                loc = rv - w0
                m = (loc >= 0) & (loc < wlim)
                mi = m.astype(jnp.int32)
                pos = cnt2 + plsc.cumsum(mi) - 1
                eid = o * IDS_BLOCK + ich * 16 + iota16
                plsc.store_scatter(ebuf, [pos], eid, mask=m)
                plsc.store_scatter(rbuf, [pos], loc, mask=m)
                return cnt2 + jnp.sum(mi)

            cnt = lax.fori_loop(0, IDS_BLOCK // 16, cbody, cnt)
            do = cnt >= CAP
            drain(jnp.where(do, cnt, 0))
            return jnp.where(do, jnp.int32(0), cnt)

        cnt = lax.fori_loop(0, N_SCAN, sbody, jnp.int32(0))
        drain(cnt)

        @pl.when(w < N_BIG)
        def _():
            pltpu.sync_copy(acc.at[pl.ds(0, W_BIG)],
                            out_hbm.at[pl.ds(w0, W_BIG)])

        @pl.when(w >= N_BIG)
        def _():
            pltpu.sync_copy(acc.at[pl.ds(0, W_SMALL)],
                            out_hbm.at[pl.ds(w0, W_SMALL)])

        return pcarry

    lax.fori_loop(0, 2, pbody, 0)


def kernel(vectors, node_feats, radial_embedding, receivers, W_proj, W_r,
           b_r, ln_g, ln_b):
    vecs2 = vectors.reshape(E_TOTAL, 3)
    rad2 = radial_embedding.reshape(E_TOTAL, N_RADIAL)
    recv = receivers.reshape(E_TOTAL).astype(jnp.int32)
    w2 = jnp.repeat(W_proj, SH_DIM, axis=1)
    t = jnp.tile(jnp.eye(SH_DIM, dtype=jnp.float32), (1, TP_CH))
    msgs = _messages_tc(vecs2, rad2, node_feats, w2, t, W_r,
                        b_r.reshape(1, MSG_DIM), ln_g.reshape(1, MSG_DIM),
                        ln_b.reshape(1, MSG_DIM))
    return _make_scatter_sc()(msgs, recv)


# speedup vs baseline: 1.2130x; 1.2130x over previous
import functools

import jax
import jax.numpy as jnp
from jax import lax
from jax.experimental import pallas as pl
from jax.experimental.pallas import tpu as pltpu
from jax.experimental.pallas import tpu_sc as plsc

N_NODES = 10000
DEG = 16
D_FEAT = 128
N_RADIAL = 8
TP_CH = 32
SH_DIM = 8
MSG_DIM = D_FEAT + TP_CH * SH_DIM
E_TOTAL = N_NODES * DEG

B_NODES = 200
B_EDGES = B_NODES * DEG
N_BLOCKS = N_NODES // B_NODES

_SQ3 = 3.0 ** 0.5
_SQ15 = 15.0 ** 0.5
_SQ5H = (5.0 ** 0.5) / 2.0
_SQ15H = _SQ15 / 2.0
_SCALE = 0.25


def _tc_body(v_ref, r_ref, feats_ref, w2_ref, t_ref, wr_ref, br_ref, g_ref,
             b_ref, out_ref):
    feats = feats_ref[...]
    p2 = jnp.dot(feats, w2_ref[...], preferred_element_type=jnp.float32)
    p2e = jnp.reshape(jnp.broadcast_to(p2[:, None, :], (B_NODES, DEG, 2 * D_FEAT)),
                      (B_EDGES, 2 * D_FEAT))
    fe = jnp.reshape(jnp.broadcast_to(feats[:, None, :], (B_NODES, DEG, D_FEAT)),
                     (B_EDGES, D_FEAT))
    v = v_ref[...]
    x = v[:, 0:1]
    y = v[:, 1:2]
    z = v[:, 2:3]
    inv = 1.0 / (jnp.sqrt(x * x + y * y + z * z) + 1e-9)
    x = x * inv
    y = y * inv
    z = z * inv
    sh8 = jnp.concatenate([
        _SQ3 * x, _SQ3 * y, _SQ3 * z,
        _SQ15 * x * y, _SQ15 * y * z, _SQ5H * (3.0 * z * z - 1.0),
        _SQ15 * x * z, _SQ15H * (x * x - y * y),
    ], axis=1)
    shm = jnp.dot(sh8, t_ref[...], preferred_element_type=jnp.float32)
    radial = jnp.dot(r_ref[...], wr_ref[...],
                     preferred_element_type=jnp.float32) + br_ref[...]
    mu = jnp.mean(radial, axis=1, keepdims=True)
    d = radial - mu
    var = jnp.mean(d * d, axis=1, keepdims=True)
    radial = (d * lax.rsqrt(var + 1e-6) * g_ref[...] + b_ref[...]) * _SCALE
    out_ref[...] = jnp.concatenate(
        [fe * radial[:, :D_FEAT], p2e * shm * radial[:, D_FEAT:]], axis=1)


def _messages_tc(vecs2, rad2, node_feats, w2, t, wr, br2, g2, b2):
    return pl.pallas_call(
        _tc_body,
        grid=(N_BLOCKS,),
        in_specs=[
            pl.BlockSpec((B_EDGES, 3), lambda i: (i, 0)),
            pl.BlockSpec((B_EDGES, N_RADIAL), lambda i: (i, 0)),
            pl.BlockSpec((B_NODES, D_FEAT), lambda i: (i, 0)),
            pl.BlockSpec((D_FEAT, 2 * D_FEAT), lambda i: (0, 0)),
            pl.BlockSpec((SH_DIM, 2 * D_FEAT), lambda i: (0, 0)),
            pl.BlockSpec((N_RADIAL, MSG_DIM), lambda i: (0, 0)),
            pl.BlockSpec((1, MSG_DIM), lambda i: (0, 0)),
            pl.BlockSpec((1, MSG_DIM), lambda i: (0, 0)),
            pl.BlockSpec((1, MSG_DIM), lambda i: (0, 0)),
        ],
        out_specs=pl.BlockSpec((B_EDGES, MSG_DIM), lambda i: (i, 0)),
        out_shape=jax.ShapeDtypeStruct((E_TOTAL, MSG_DIM), jnp.float32),
    )(vecs2, rad2, node_feats, w2, t, wr, br2, g2, b2)


ACC_ROWS = 168
DUMP_ROW = 160
W_BIG = 160
W_SMALL = 152
N_BIG = 34
CAP = 512
IDS_BLOCK = 1600
EBUF = CAP + IDS_BLOCK + 32
N_SCAN = E_TOTAL // IDS_BLOCK
GCHUNK = 32


@functools.cache
def _make_scatter_sc():
    mesh = plsc.VectorSubcoreMesh(core_axis_name="c", subcore_axis_name="s",
                                  num_cores=2, num_subcores=16)
    return pl.kernel(
        _sc_body,
        out_type=jax.ShapeDtypeStruct((N_NODES, MSG_DIM), jnp.float32),
        mesh=mesh,
        compiler_params=pltpu.CompilerParams(needs_layout_passes=False),
        scratch_types=[
            pltpu.VMEM((ACC_ROWS, MSG_DIM), jnp.float32),
            pltpu.VMEM((GCHUNK, MSG_DIM), jnp.float32),
            pltpu.VMEM((EBUF,), jnp.int32),
            pltpu.VMEM((EBUF,), jnp.int32),
            pltpu.VMEM((IDS_BLOCK,), jnp.int32),
        ],
    )


def _sc_body(msgs_hbm, recv_hbm, out_hbm, acc, mbuf, ebuf, rbuf, idbuf):
    cid = lax.axis_index("c")
    sid = lax.axis_index("s")
    t = sid * 2 + cid

    iota16 = lax.iota(jnp.int32, 16)
    zero16f = jnp.zeros((16,), jnp.float32)
    zero16i = jnp.zeros((16,), jnp.int32)
    dump16 = jnp.full((16,), DUMP_ROW, jnp.int32)

    def drain(n):
        mpad = jnp.broadcast_to(n > 0, (16,))
        plsc.store_scatter(rbuf, [n + iota16], dump16, mask=mpad)
        plsc.store_scatter(rbuf, [n + 16 + iota16], dump16, mask=mpad)
        nch = (n + (GCHUNK - 1)) // GCHUNK

        def dbody(dk, carry):
            pltpu.sync_copy(msgs_hbm.at[ebuf.at[pl.ds(dk * GCHUNK, GCHUNK)]],
                            mbuf)
            rows = [jnp.max(plsc.load_gather(
                rbuf, [jnp.full((16,), dk * GCHUNK + r, jnp.int32)]))
                for r in range(GCHUNK)]
            for r in range(GCHUNK):
                for cg in range(MSG_DIM // 16):
                    plsc.addupdate(acc.at[rows[r], pl.ds(cg * 16, 16)],
                                   mbuf[r, pl.ds(cg * 16, 16)])
            return carry

        lax.fori_loop(0, nch, dbody, 0)

    def pbody(p, pcarry):
        w = t + 32 * p
        w0 = jnp.where(w < N_BIG, w * W_BIG,
                       N_BIG * W_BIG + (w - N_BIG) * W_SMALL).astype(jnp.int32)
        w0 = pl.multiple_of(w0, 8)
        wlim = jnp.where(w < N_BIG, W_BIG, W_SMALL).astype(jnp.int32)

        def zbody(r, zc):
            for cg in range(MSG_DIM // 16):
                acc[r, pl.ds(cg * 16, 16)] = zero16f
            return zc

        lax.fori_loop(0, ACC_ROWS, zbody, 0)

        def ezbody(k, zc):
            plsc.store_scatter(ebuf, [k * 16 + iota16], zero16i)
            return zc

        lax.fori_loop(0, EBUF // 16, ezbody, 0)

        def sbody(o, cnt):
            pltpu.sync_copy(recv_hbm.at[pl.ds(o * IDS_BLOCK, IDS_BLOCK)],
                            idbuf)
            cntv = jnp.full((16,), 0, jnp.int32) + cnt

            def cbody(j, cv):
                ms, css, pcs, eids, locs = [], [], [], [], []
                for g in range(4):
                    base = j * 64 + g * 16
                    rv = plsc.load_gather(idbuf, [base + iota16])
                    loc = rv - w0
                    m = (loc >= 0) & (loc < wlim)
                    ms.append(m)
                    locs.append(loc)
                    css.append(plsc.cumsum(m.astype(jnp.int32)))
                    pcs.append(plsc.all_reduce_population_count(m))
                    eids.append(o * IDS_BLOCK + base + iota16)
                pref = cv
                for g in range(4):
                    pos = pref + css[g] - 1
                    plsc.store_scatter(ebuf, [pos], eids[g], mask=ms[g])
                    plsc.store_scatter(rbuf, [pos], locs[g], mask=ms[g])
                    pref = pref + pcs[g]
                return pref

            cntv = lax.fori_loop(0, IDS_BLOCK // 64, cbody, cntv)
            cnt = jnp.max(cntv)
            do = cnt >= CAP
            drain(jnp.where(do, cnt, 0))
            return jnp.where(do, jnp.int32(0), cnt)

        cnt = lax.fori_loop(0, N_SCAN, sbody, jnp.int32(0))
        drain(cnt)

        @pl.when(w < N_BIG)
        def _():
            pltpu.sync_copy(acc.at[pl.ds(0, W_BIG)],
                            out_hbm.at[pl.ds(w0, W_BIG)])

        @pl.when(w >= N_BIG)
        def _():
            pltpu.sync_copy(acc.at[pl.ds(0, W_SMALL)],
                            out_hbm.at[pl.ds(w0, W_SMALL)])

        return pcarry

    lax.fori_loop(0, 2, pbody, 0)


def kernel(vectors, node_feats, radial_embedding, receivers, W_proj, W_r,
           b_r, ln_g, ln_b):
    vecs2 = vectors.reshape(E_TOTAL, 3)
    rad2 = radial_embedding.reshape(E_TOTAL, N_RADIAL)
    recv = receivers.reshape(E_TOTAL).astype(jnp.int32)
    w2 = jnp.repeat(W_proj, SH_DIM, axis=1)
    t = jnp.tile(jnp.eye(SH_DIM, dtype=jnp.float32), (1, TP_CH))
    msgs = _messages_tc(vecs2, rad2, node_feats, w2, t, W_r,
                        b_r.reshape(1, MSG_DIM), ln_g.reshape(1, MSG_DIM),
                        ln_b.reshape(1, MSG_DIM))
    return _make_scatter_sc()(msgs, recv)
